# baseline (speedup 1.0000x reference)
import functools

import jax
import jax.numpy as jnp
from jax.experimental import pallas as pl
from jax.experimental.pallas import tpu as pltpu

B, S, D = 4, 2048, 1024
ITERS = 4
NUM_LAYERS = 4
LN_EPS = 1e-5
T = B * S
TM = 512


def _fused_kernel(h_ref, m_ref, sm_ref, bm_ref, st_ref, bt_ref, oh_ref,
                  wrm_ref, wrt_ref, wl_ref, out_ref, probs_ref):
    h = h_ref[...]
    hm = h * m_ref[...]
    dp = float(D + ITERS)
    oh = oh_ref[...]
    oh_s1 = jnp.sum(oh)
    oh_s2 = jnp.sum(oh * oh)
    s1 = jnp.sum(hm, axis=1, keepdims=True)
    s2 = jnp.sum(hm * hm, axis=1, keepdims=True)
    mean = (s1 + oh_s1) / dp
    var = (s2 + oh_s2) / dp - mean * mean
    inv = jax.lax.rsqrt(var + LN_EPS)
    xs = (hm - mean) * inv * sm_ref[...] + bm_ref[...]
    x_tail = (oh - mean) * inv * st_ref[...] + bt_ref[...]
    logits = (jnp.dot(xs, wrm_ref[...], preferred_element_type=jnp.float32)
              + jnp.dot(x_tail, wrt_ref[...],
                        preferred_element_type=jnp.float32))
    lmax = jnp.max(logits, axis=1, keepdims=True)
    e = jnp.exp(logits - lmax)
    p = e / jnp.sum(e, axis=1, keepdims=True)
    probs_ref[...] = p
    hb = hm.astype(jnp.bfloat16)
    y = jnp.dot(hb, wl_ref[...], preferred_element_type=jnp.float32)
    acc = None
    for l in range(NUM_LAYERS):
        wy = p[:, l:l + 1] * y[:, l * D:(l + 1) * D]
        acc = wy if acc is None else acc + wy
    out_ref[...] = acc


@functools.partial(jax.jit, static_argnames=())
def kernel(hidden_states, attention_mask, ln_scale, ln_bias, W_router,
           W_layers, iteration):
    h2 = hidden_states.reshape(T, D)
    mask2 = attention_mask.reshape(T, 1).astype(jnp.float32)
    scale_main = ln_scale[:D].reshape(1, D)
    scale_tail = ln_scale[D:].reshape(1, ITERS)
    bias_main = ln_bias[:D].reshape(1, D)
    bias_tail = ln_bias[D:].reshape(1, ITERS)
    oh_row = jax.nn.one_hot(iteration, ITERS, dtype=jnp.float32).reshape(1, ITERS)
    wrm = W_router[:, :D].T
    wrt = W_router[:, D:].T
    wl_bf = (W_layers.astype(jnp.bfloat16)
             .transpose(1, 0, 2).reshape(D, NUM_LAYERS * D))

    full = lambda shp: pl.BlockSpec(shp, lambda i: (0,) * len(shp))
    grid = (T // TM,)
    out, probs = pl.pallas_call(
        _fused_kernel,
        grid=grid,
        in_specs=[
            pl.BlockSpec((TM, D), lambda i: (i, 0)),
            pl.BlockSpec((TM, 1), lambda i: (i, 0)),
            full((1, D)), full((1, D)),
            full((1, ITERS)), full((1, ITERS)), full((1, ITERS)),
            full((D, NUM_LAYERS)), full((ITERS, NUM_LAYERS)),
            full((D, NUM_LAYERS * D)),
        ],
        out_specs=[
            pl.BlockSpec((TM, D), lambda i: (i, 0)),
            pl.BlockSpec((TM, NUM_LAYERS), lambda i: (i, 0)),
        ],
        out_shape=[
            jax.ShapeDtypeStruct((T, D), jnp.float32),
            jax.ShapeDtypeStruct((T, NUM_LAYERS), jnp.float32),
        ],
        compiler_params=pltpu.CompilerParams(
            dimension_semantics=("parallel",),
        ),
    )(h2, mask2, scale_main, bias_main, scale_tail, bias_tail, oh_row,
      wrm, wrt, wl_bf)
    return out.reshape(B, S, D), probs.reshape(B, S, NUM_LAYERS)


# speedup vs baseline: 1.1399x; 1.1399x over previous
import functools

import jax
import jax.numpy as jnp
from jax.experimental import pallas as pl
from jax.experimental.pallas import tpu as pltpu

B, S, D = 4, 2048, 1024
ITERS = 4
NUM_LAYERS = 4
LN_EPS = 1e-5
T = B * S
TM = 1024


def _fused_kernel(h_ref, m_ref, sm_ref, bm_ref, st_ref, bt_ref, oh_ref,
                  wrm_ref, wrt_ref, wl_ref, out_ref, probs_ref):
    h = h_ref[...]
    hm = h * m_ref[...]
    dp = float(D + ITERS)
    oh = oh_ref[...]
    oh_s1 = jnp.sum(oh)
    oh_s2 = jnp.sum(oh * oh)
    s1 = jnp.sum(hm, axis=1, keepdims=True)
    s2 = jnp.sum(hm * hm, axis=1, keepdims=True)
    mean = (s1 + oh_s1) / dp
    var = (s2 + oh_s2) / dp - mean * mean
    inv = jax.lax.rsqrt(var + LN_EPS)
    xs = (hm - mean) * inv * sm_ref[...] + bm_ref[...]
    x_tail = (oh - mean) * inv * st_ref[...] + bt_ref[...]
    logits = (jnp.dot(xs, wrm_ref[...], preferred_element_type=jnp.float32)
              + jnp.dot(x_tail, wrt_ref[...],
                        preferred_element_type=jnp.float32))
    lmax = jnp.max(logits, axis=1, keepdims=True)
    e = jnp.exp(logits - lmax)
    p = e / jnp.sum(e, axis=1, keepdims=True)
    probs_ref[...] = p
    hb = hm.astype(jnp.bfloat16)
    acc = None
    for l in range(NUM_LAYERS):
        y = jnp.dot(hb, wl_ref[l], preferred_element_type=jnp.float32)
        wy = p[:, l:l + 1] * y
        acc = wy if acc is None else acc + wy
    out_ref[...] = acc


@functools.partial(jax.jit, static_argnames=())
def kernel(hidden_states, attention_mask, ln_scale, ln_bias, W_router,
           W_layers, iteration):
    h2 = hidden_states.reshape(T, D)
    mask2 = attention_mask.reshape(T, 1).astype(jnp.float32)
    scale_main = ln_scale[:D].reshape(1, D)
    scale_tail = ln_scale[D:].reshape(1, ITERS)
    bias_main = ln_bias[:D].reshape(1, D)
    bias_tail = ln_bias[D:].reshape(1, ITERS)
    oh_row = jax.nn.one_hot(iteration, ITERS, dtype=jnp.float32).reshape(1, ITERS)
    wrm = W_router[:, :D].T
    wrt = W_router[:, D:].T
    wl_bf = W_layers.astype(jnp.bfloat16)

    full = lambda shp: pl.BlockSpec(shp, lambda i: (0,) * len(shp))
    grid = (T // TM,)
    out, probs = pl.pallas_call(
        _fused_kernel,
        grid=grid,
        in_specs=[
            pl.BlockSpec((TM, D), lambda i: (i, 0)),
            pl.BlockSpec((TM, 1), lambda i: (i, 0)),
            full((1, D)), full((1, D)),
            full((1, ITERS)), full((1, ITERS)), full((1, ITERS)),
            full((D, NUM_LAYERS)), full((ITERS, NUM_LAYERS)),
            full((NUM_LAYERS, D, D)),
        ],
        out_specs=[
            pl.BlockSpec((TM, D), lambda i: (i, 0)),
            pl.BlockSpec((TM, NUM_LAYERS), lambda i: (i, 0)),
        ],
        out_shape=[
            jax.ShapeDtypeStruct((T, D), jnp.float32),
            jax.ShapeDtypeStruct((T, NUM_LAYERS), jnp.float32),
        ],
        compiler_params=pltpu.CompilerParams(
            dimension_semantics=("parallel",),
        ),
    )(h2, mask2, scale_main, bias_main, scale_tail, bias_tail, oh_row,
      wrm, wrt, wl_bf)
    return out.reshape(B, S, D), probs.reshape(B, S, NUM_LAYERS)
